# baseline (speedup 1.0000x reference)
import functools

import jax
import jax.numpy as jnp
from jax import lax
from jax.experimental import pallas as pl
from jax.experimental.pallas import tpu as pltpu
from jax.experimental.pallas import tpu_sc as plsc

_D = 13
_NW = 32


def _sc_bias_kernel(CH, n_chunks, at_hbm, wvc_hbm, out_hbm, buf0, buf1, acc_buf, wv_buf):
    wid = lax.axis_index("s") * 2 + lax.axis_index("c")

    pltpu.sync_copy(wvc_hbm, wv_buf)
    wv_all = wv_buf[...]

    def _splat(d):
        idx = jnp.full((16,), d, jnp.int32)
        return wv_all.at[idx].get(mode="promise_in_bounds")

    wvecs = [_splat(d) for d in range(_D)]
    cvec = _splat(_D)

    per_w = -(-n_chunks // _NW)

    def chunk_body(t, _):
        i = t * _NW + wid

        @pl.when(i < n_chunks)
        def _():
            e0 = i * CH
            pltpu.sync_copy(at_hbm.at[pl.ds(0, 8), pl.ds(e0, CH)], buf0)
            pltpu.sync_copy(at_hbm.at[pl.ds(8, _D - 8), pl.ds(e0, CH)], buf1)

            def group(g):
                acc = cvec
                for d in range(8):
                    acc = acc + buf0[d, pl.ds(g, 16)] * wvecs[d]
                for d in range(_D - 8):
                    acc = acc + buf1[d, pl.ds(g, 16)] * wvecs[8 + d]
                acc_buf[pl.ds(g, 16)] = acc

            plsc.parallel_loop(0, CH, step=16, unroll=4)(group)
            pltpu.sync_copy(acc_buf, out_hbm.at[pl.ds(e0, CH)])

        return 0

    lax.fori_loop(0, per_w, chunk_body, 0)


def kernel(edge_attr, W_edge, b_edge, edge_index, n_nodes, batch):
    E, D = edge_attr.shape
    if E == 0:
        return jnp.zeros((0,), dtype=jnp.float32)

    wv = jnp.mean(W_edge, axis=1)
    c = jnp.mean(b_edge)
    wvc = jnp.concatenate([wv, c[None], jnp.zeros((16 - D - 1,), jnp.float32)])

    At = edge_attr.T

    CH = 2560
    while E % CH:
        CH -= 128
    n_chunks = E // CH

    mesh = plsc.VectorSubcoreMesh(
        core_axis_name="c", subcore_axis_name="s", num_cores=2, num_subcores=16
    )
    k = pl.kernel(
        functools.partial(_sc_bias_kernel, CH, n_chunks),
        mesh=mesh,
        out_type=jax.ShapeDtypeStruct((E,), jnp.float32),
        scratch_types=[
            pltpu.VMEM((8, CH), jnp.float32),
            pltpu.VMEM((_D - 8, CH), jnp.float32),
            pltpu.VMEM((CH,), jnp.float32),
            pltpu.VMEM((16,), jnp.float32),
        ],
        compiler_params=pltpu.CompilerParams(needs_layout_passes=False),
    )
    return k(At, wvc)


# speedup vs baseline: 1.8936x; 1.8936x over previous
import functools

import jax
import jax.numpy as jnp
from jax import lax
from jax.experimental import pallas as pl
from jax.experimental.pallas import tpu as pltpu
from jax.experimental.pallas import tpu_sc as plsc

_D = 13
_NW = 32


def _sc_bias_kernel(
    CH, n_chunks, at_hbm, wvc_hbm, out_hbm,
    b0a, b1a, b0b, b1b, acc_buf, wv_buf, sem_a, sem_b,
):
    wid = lax.axis_index("s") * 2 + lax.axis_index("c")
    n_t = (n_chunks - wid + _NW - 1) // _NW

    pltpu.sync_copy(wvc_hbm, wv_buf)
    wv_all = wv_buf[...]

    def _splat(d):
        idx = jnp.full((16,), d, jnp.int32)
        return wv_all.at[idx].get(mode="promise_in_bounds")

    wvecs = [_splat(d) for d in range(_D)]
    cvec = _splat(_D)

    pairs = ((b0a, b1a, sem_a), (b0b, b1b, sem_b))

    def in_copies(t, pair):
        b0, b1, sem = pair
        e0 = (t * _NW + wid) * CH
        return (
            pltpu.make_async_copy(at_hbm.at[pl.ds(0, 8), pl.ds(e0, CH)], b0, sem),
            pltpu.make_async_copy(at_hbm.at[pl.ds(8, _D - 8), pl.ds(e0, CH)], b1, sem),
        )

    @pl.when(n_t > 0)
    def _():
        for cp in in_copies(0, pairs[0]):
            cp.start()

    per_w = -(-n_chunks // _NW)
    n_outer = -(-per_w // 2)

    def outer(k, _):
        for b in range(2):
            t = k * 2 + b

            @pl.when(t + 1 < n_t)
            def _():
                for cp in in_copies(t + 1, pairs[1 - b]):
                    cp.start()

            @pl.when(t < n_t)
            def _():
                b0, b1, _sem = pairs[b]
                for cp in in_copies(t, pairs[b]):
                    cp.wait()

                def group(g):
                    acc = cvec
                    for d in range(8):
                        acc = acc + b0[d, pl.ds(g, 16)] * wvecs[d]
                    for d in range(_D - 8):
                        acc = acc + b1[d, pl.ds(g, 16)] * wvecs[8 + d]
                    acc_buf[pl.ds(g, 16)] = acc

                plsc.parallel_loop(0, CH, step=16, unroll=4)(group)
                e0 = (t * _NW + wid) * CH
                pltpu.sync_copy(acc_buf, out_hbm.at[pl.ds(e0, CH)])

        return 0

    lax.fori_loop(0, n_outer, outer, 0)


def kernel(edge_attr, W_edge, b_edge, edge_index, n_nodes, batch):
    E, D = edge_attr.shape
    if E == 0:
        return jnp.zeros((0,), dtype=jnp.float32)

    wv = jnp.mean(W_edge, axis=1)
    c = jnp.mean(b_edge)
    wvc = jnp.concatenate([wv, c[None], jnp.zeros((16 - D - 1,), jnp.float32)])

    At = edge_attr.T

    CH = 3200
    while E % CH:
        CH -= 128
    n_chunks = E // CH

    mesh = plsc.VectorSubcoreMesh(
        core_axis_name="c", subcore_axis_name="s", num_cores=2, num_subcores=16
    )
    k = pl.kernel(
        functools.partial(_sc_bias_kernel, CH, n_chunks),
        mesh=mesh,
        out_type=jax.ShapeDtypeStruct((E,), jnp.float32),
        scratch_types=[
            pltpu.VMEM((8, CH), jnp.float32),
            pltpu.VMEM((_D - 8, CH), jnp.float32),
            pltpu.VMEM((8, CH), jnp.float32),
            pltpu.VMEM((_D - 8, CH), jnp.float32),
            pltpu.VMEM((CH,), jnp.float32),
            pltpu.VMEM((16,), jnp.float32),
            pltpu.SemaphoreType.DMA,
            pltpu.SemaphoreType.DMA,
        ],
        compiler_params=pltpu.CompilerParams(needs_layout_passes=False),
    )
    return k(At, wvc)
